# baseline (device time: 45145 ns/iter reference)
import jax
import jax.numpy as jnp
from jax import lax
from jax.experimental import pallas as pl
from jax.experimental.pallas import tpu as pltpu

N_DEV = 4
B = 2
S = 512
D = 768
H_LOCAL = 4
DH = 96
SCALE = 0.10206207261596577
EPS = 1e-5
ROWS = B * S
U = 256
N_U = ROWS // U
DHP = 128
F32 = jnp.float32
BF16 = jnp.bfloat16
FP8 = jnp.float8_e4m3fn


def _ln(h):
    m = jnp.mean(h, axis=-1, keepdims=True)
    v = jnp.mean(h * h, axis=-1, keepdims=True) - m * m
    return (h - m) * lax.rsqrt(v + EPS)


def _body(x_ref, wq_ref, wk_ref, wv_ref, wo_ref, temb_ref, wmod_ref,
          wff1_ref, wff2_ref, out_ref, comm_ref, send_sems, recv_sems):
    my = lax.axis_index("i")
    left = lax.rem(my + N_DEV - 1, N_DEV)
    right = lax.rem(my + 1, N_DEV)
    par = lax.rem(my, 2)
    p_a = my + 1 - 2 * par
    p_b = lax.rem(my + 3 + 2 * par, N_DEV)

    mod = jnp.dot(temb_ref[:, :], wmod_ref[:, :],
                  preferred_element_type=F32)

    barrier_sem = pltpu.get_barrier_semaphore()
    for nbr in (left, right):
        pl.semaphore_signal(
            barrier_sem, inc=1,
            device_id=(nbr,), device_id_type=pl.DeviceIdType.MESH,
        )
    pl.semaphore_wait(barrier_sem, 2)

    def exchange(src_slot, dst_slot, partner, sem):
        rdma = pltpu.make_async_remote_copy(
            src_ref=comm_ref.at[src_slot],
            dst_ref=comm_ref.at[dst_slot],
            send_sem=send_sems.at[sem],
            recv_sem=recv_sems.at[sem],
            device_id=(partner,),
            device_id_type=pl.DeviceIdType.MESH,
        )
        rdma.start()
        return rdma

    def _base(u, r):
        return 16 * r + 4 * u

    def _sem(u, r, stage):
        return 8 * r + 2 * u + stage

    def _partners(u):
        return (p_a, p_b) if u % 2 == 0 else (p_b, p_a)

    def stage1(u, r, val_f32):
        b = _base(u, r)
        comm_ref[b] = val_f32.astype(FP8)
        return exchange(b, b + 1, _partners(u)[0], _sem(u, r, 0))

    def stage2(u, r, val_f32):
        b = _base(u, r)
        s = val_f32.astype(BF16) + comm_ref[b + 1].astype(BF16)
        comm_ref[b + 2] = s.astype(FP8)
        return s, exchange(b + 2, b + 3, _partners(u)[1], _sem(u, r, 1))

    def finish(u, r, pair_sum):
        b = _base(u, r)
        return pair_sum.astype(F32) + comm_ref[b + 3].astype(F32)

    def mod_slice(idx, b):
        return mod[b:b + 1, idx * D:(idx + 1) * D]

    wq = wq_ref[:, :]
    wk = wk_ref[:, :]
    wv = wv_ref[:, :]
    wo = wo_ref[:, :]
    wff1 = wff1_ref[:, :]
    wff2 = wff2_ref[:, :]

    def qkv(b):
        x0b = x_ref[pl.ds(b * S, S), :]
        xm = _ln(x0b) * (1.0 + mod_slice(0, b)) + mod_slice(1, b)
        xmb = xm.astype(BF16)
        qb = jnp.dot(xmb, wq, preferred_element_type=F32).astype(BF16)
        kb = jnp.dot(xmb, wk, preferred_element_type=F32).astype(BF16)
        vb = jnp.dot(xmb, wv, preferred_element_type=F32).astype(BF16)
        return qb, kb, vb

    def attn_unit(qkv_t, half):
        qb, kb, vb = qkv_t
        qs = qb[half * U:(half + 1) * U, :]
        heads = []
        for hh in range(H_LOCAL):
            sl = slice(hh * DHP, (hh + 1) * DHP)
            qh, kh, vh = qs[:, sl], kb[:, sl], vb[:, sl]
            s = lax.dot_general(
                qh, kh, (((1,), (1,)), ((), ())),
                preferred_element_type=F32,
            ).astype(BF16)
            p = jnp.exp(s)
            l = jnp.sum(p.astype(F32), axis=-1, keepdims=True)
            o = jnp.dot(p, vh, preferred_element_type=F32) / l
            heads.append(o)
        attn = jnp.concatenate(heads, axis=1).astype(BF16)
        return jnp.dot(attn, wo, preferred_element_type=F32)

    def ffn_unit(x1u, b):
        xm2 = _ln(x1u) * (1.0 + mod_slice(3, b)) + mod_slice(4, b)
        hb = jnp.dot(xm2.astype(BF16), wff1, preferred_element_type=F32)
        hb = hb / (1.0 + jnp.exp(-hb))
        return jnp.dot(hb.astype(BF16), wff2, preferred_element_type=F32)

    def x0_rows(u):
        return x_ref[pl.ds(u * U, U), :]

    e1 = [None] * N_U
    e2 = [None] * N_U
    ps = [None] * N_U
    f1 = [None] * N_U
    f2 = [None] * N_U
    ts = [None] * N_U
    x1 = [None] * N_U
    q = [None] * N_U

    kv0 = qkv(0)
    p0 = attn_unit(kv0, 0)
    e1[0] = stage1(0, 0, p0)
    p1 = attn_unit(kv0, 1)
    e1[1] = stage1(1, 0, p1)
    kv1 = qkv(1)
    e1[0].wait()
    ps[0], e2[0] = stage2(0, 0, p0)
    p2 = attn_unit(kv1, 0)
    e1[2] = stage1(2, 0, p2)
    e1[1].wait()
    ps[1], e2[1] = stage2(1, 0, p1)
    p3 = attn_unit(kv1, 1)
    e1[3] = stage1(3, 0, p3)

    e2[0].wait()
    x1[0] = x0_rows(0) + mod_slice(2, 0) * finish(0, 0, ps[0])
    q[0] = ffn_unit(x1[0], 0)
    f1[0] = stage1(0, 1, q[0])
    e1[2].wait()
    ps[2], e2[2] = stage2(2, 0, p2)
    e2[1].wait()
    x1[1] = x0_rows(1) + mod_slice(2, 0) * finish(1, 0, ps[1])
    q[1] = ffn_unit(x1[1], 0)
    f1[1] = stage1(1, 1, q[1])
    e1[3].wait()
    ps[3], e2[3] = stage2(3, 0, p3)
    e2[2].wait()
    x1[2] = x0_rows(2) + mod_slice(2, 1) * finish(2, 0, ps[2])
    q[2] = ffn_unit(x1[2], 1)
    f1[2] = stage1(2, 1, q[2])
    f1[0].wait()
    ts[0], f2[0] = stage2(0, 1, q[0])
    e2[3].wait()
    x1[3] = x0_rows(3) + mod_slice(2, 1) * finish(3, 0, ps[3])
    q[3] = ffn_unit(x1[3], 1)
    f1[3] = stage1(3, 1, q[3])
    f1[1].wait()
    ts[1], f2[1] = stage2(1, 1, q[1])
    f2[0].wait()
    out_ref[pl.ds(0, U), :] = x1[0] + mod_slice(5, 0) * finish(0, 1, ts[0])
    f1[2].wait()
    ts[2], f2[2] = stage2(2, 1, q[2])
    f2[1].wait()
    out_ref[pl.ds(U, U), :] = x1[1] + mod_slice(5, 0) * finish(1, 1, ts[1])
    f1[3].wait()
    ts[3], f2[3] = stage2(3, 1, q[3])
    f2[2].wait()
    out_ref[pl.ds(2 * U, U), :] = x1[2] + mod_slice(5, 1) * finish(2, 1, ts[2])
    f2[3].wait()
    out_ref[pl.ds(3 * U, U), :] = x1[3] + mod_slice(5, 1) * finish(3, 1, ts[3])


def _pad_cols(w):
    w4 = w.astype(BF16).reshape(D, H_LOCAL, DH)
    return jnp.pad(w4, ((0, 0), (0, 0), (0, DHP - DH))).reshape(
        D, H_LOCAL * DHP)


def _pad_rows(w):
    w4 = w.astype(BF16).reshape(H_LOCAL, DH, D)
    return jnp.pad(w4, ((0, 0), (0, DHP - DH), (0, 0))).reshape(
        H_LOCAL * DHP, D)


def kernel(x, Wq, Wk, Wv, Wo, t_emb, W_mod, W_ff1, W_ff2):
    x2d = x.reshape(ROWS, D)
    out = pl.pallas_call(
        _body,
        out_shape=jax.ShapeDtypeStruct((ROWS, D), F32),
        in_specs=[pl.BlockSpec(memory_space=pltpu.VMEM)] * 9,
        out_specs=pl.BlockSpec(memory_space=pltpu.VMEM),
        scratch_shapes=[
            pltpu.VMEM((32, U, D), FP8),
            pltpu.SemaphoreType.DMA((16,)),
            pltpu.SemaphoreType.DMA((16,)),
        ],
        compiler_params=pltpu.CompilerParams(collective_id=0),
    )(x2d, _pad_cols(Wq * SCALE), _pad_cols(Wk), _pad_cols(Wv),
      _pad_rows(Wo), t_emb, W_mod, W_ff1.astype(BF16), W_ff2.astype(BF16))
    return out.reshape(B, S, D)
